# Initial kernel scaffold; baseline (speedup 1.0000x reference)
import jax
import jax.numpy as jnp
from jax.experimental import pallas as pl


def kernel(x, edge_index, W1, b1, g1, be1, W2, b2, g2, be2, W3, b3):
    raise NotImplementedError("write your pallas kernel here")


# speedup vs baseline: 6.5052x; 6.5052x over previous
import functools

import jax
import jax.numpy as jnp
from jax import lax
from jax.experimental import pallas as pl
from jax.experimental.pallas import tpu as pltpu
from jax.experimental.pallas import tpu_sc as plsc

N = 10000
E = 320000
F_IN = 128
HID = 128
OUT = 11
OUTP = 16

NCORES = 2
NSUB = 16
NW = NCORES * NSUB
CH = 128
KPC = 80
EPAD = NW * KPC * CH
ACC_ROWS = 10240
PAD_DST = N
ZROWS = 32


def _sc_mesh():
    return plsc.VectorSubcoreMesh(core_axis_name="c", subcore_axis_name="s")


def _degree(dst3):
    @functools.partial(
        pl.kernel,
        mesh=_sc_mesh(),
        out_type=jax.ShapeDtypeStruct((2 * N, HID), jnp.float32),
        scratch_types=[
            pltpu.VMEM((KPC, CH), jnp.int32),
            pltpu.VMEM((CH, HID), jnp.float32),
            pltpu.VMEM((ZROWS, HID), jnp.float32),
            pltpu.VMEM_SHARED((ACC_ROWS, HID), jnp.float32),
        ],
    )
    def kern(dst_hbm, out_hbm, didx, ones, zbuf, acc):
        c = lax.axis_index("c")
        s = lax.axis_index("s")
        wid = s * NCORES + c

        one16 = jnp.ones((16,), jnp.float32)
        zero16 = jnp.zeros((16,), jnp.float32)
        for i in range(CH):
            for j in range(HID // 16):
                ones[i, pl.ds(j * 16, 16)] = one16
        for i in range(ZROWS):
            for j in range(HID // 16):
                zbuf[i, pl.ds(j * 16, 16)] = zero16

        rps = ACC_ROWS // NSUB
        for i in range(rps // ZROWS):
            pltpu.sync_copy(zbuf, acc.at[pl.ds(s * rps + i * ZROWS, ZROWS)])
        plsc.subcore_barrier()

        pltpu.sync_copy(dst_hbm.at[wid], didx)

        def body(k, carry):
            pltpu.sync_copy(ones, acc.at[didx.at[k]], add=True)
            return carry

        lax.fori_loop(0, KPC, body, 0)

        plsc.subcore_barrier()
        pltpu.sync_copy(acc.at[pl.ds(s * 624, 624)],
                        out_hbm.at[pl.ds(c * N + s * 624, 624)])
        @pl.when(s == 0)
        def _():
            pltpu.sync_copy(acc.at[pl.ds(9984, 16)],
                            out_hbm.at[pl.ds(c * N + 9984, 16)])

    return kern(dst3)


def _propagate(hp, src3, dst3, D):
    @functools.partial(
        pl.kernel,
        mesh=_sc_mesh(),
        out_type=jax.ShapeDtypeStruct((2 * N, D), jnp.float32),
        scratch_types=[
            pltpu.VMEM((KPC, CH), jnp.int32),
            pltpu.VMEM((KPC, CH), jnp.int32),
            pltpu.VMEM((CH, D), jnp.float32),
            pltpu.VMEM((ZROWS, D), jnp.float32),
            pltpu.VMEM_SHARED((ACC_ROWS, D), jnp.float32),
            pltpu.SemaphoreType.DMA,
        ],
    )
    def kern(hp_hbm, src_hbm, dst_hbm, out_hbm, sidx, didx, rows, zbuf, acc, sem):
        c = lax.axis_index("c")
        s = lax.axis_index("s")
        wid = s * NCORES + c

        zero16 = jnp.zeros((16,), jnp.float32)
        for i in range(ZROWS):
            for j in range(D // 16):
                zbuf[i, pl.ds(j * 16, 16)] = zero16

        rps = ACC_ROWS // NSUB
        for i in range(rps // ZROWS):
            pltpu.sync_copy(zbuf, acc.at[pl.ds(s * rps + i * ZROWS, ZROWS)])
        plsc.subcore_barrier()

        pltpu.sync_copy(src_hbm.at[wid], sidx)
        pltpu.sync_copy(dst_hbm.at[wid], didx)

        def body(k, carry):
            pltpu.async_copy(hp_hbm.at[sidx.at[k]], rows, sem).wait()
            pltpu.sync_copy(rows, acc.at[didx.at[k]], add=True)
            return carry

        lax.fori_loop(0, KPC, body, 0)

        plsc.subcore_barrier()
        pltpu.sync_copy(acc.at[pl.ds(s * 624, 624)],
                        out_hbm.at[pl.ds(c * N + s * 624, 624)])
        @pl.when(s == 0)
        def _():
            pltpu.sync_copy(acc.at[pl.ds(9984, 16)],
                            out_hbm.at[pl.ds(c * N + 9984, 16)])

    return kern(hp, src3, dst3)


def _dinv_from(degp_ref):
    deg = degp_ref[0:N, 0] + degp_ref[N:2 * N, 0] + 1.0
    return lax.rsqrt(deg)


def _tc_first(x, W1, degp):
    def body(x_ref, w_ref, degp_ref, o_ref):
        dinv = _dinv_from(degp_ref)[:, None]
        h = jnp.dot(x_ref[...], w_ref[...], preferred_element_type=jnp.float32)
        o_ref[...] = h * dinv

    return pl.pallas_call(
        body,
        out_shape=jax.ShapeDtypeStruct((N, HID), jnp.float32),
    )(x, W1, degp)


def _tc_mid(partials, hp, degp, b, g, be, Wn, Dn):
    def body(p_ref, hp_ref, degp_ref, b_ref, g_ref, be_ref, w_ref, o_ref):
        dinv = _dinv_from(degp_ref)[:, None]
        z = dinv * (p_ref[0:N, :] + p_ref[N:2 * N, :] + hp_ref[...]) + b_ref[...]
        mu = jnp.mean(z, axis=0, keepdims=True)
        var = jnp.mean((z - mu) * (z - mu), axis=0, keepdims=True)
        zn = (z - mu) * lax.rsqrt(var + 1e-5) * g_ref[...] + be_ref[...]
        h = jnp.maximum(zn, 0.0)
        o_ref[...] = jnp.dot(h, w_ref[...], preferred_element_type=jnp.float32) * dinv

    return pl.pallas_call(
        body,
        out_shape=jax.ShapeDtypeStruct((N, Dn), jnp.float32),
    )(partials, hp, degp, b, g, be, Wn)


def _tc_mid_nomm(partials, hp, degp, b, g, be):
    def body(p_ref, hp_ref, degp_ref, b_ref, g_ref, be_ref, o_ref):
        dinv = _dinv_from(degp_ref)[:, None]
        z = dinv * (p_ref[0:N, :] + p_ref[N:2 * N, :] + hp_ref[...]) + b_ref[...]
        mu = jnp.mean(z, axis=0, keepdims=True)
        var = jnp.mean((z - mu) * (z - mu), axis=0, keepdims=True)
        zn = (z - mu) * lax.rsqrt(var + 1e-5) * g_ref[...] + be_ref[...]
        o_ref[...] = jnp.maximum(zn, 0.0) * dinv

    return pl.pallas_call(
        body,
        out_shape=jax.ShapeDtypeStruct((N, HID), jnp.float32),
    )(partials, hp, degp, b, g, be)


def _tc_final(partials, hp, degp, W, b):
    def body(p_ref, hp_ref, degp_ref, w_ref, b_ref, o_ref):
        dinv = _dinv_from(degp_ref)[:, None]
        q = dinv * (p_ref[0:N, :] + p_ref[N:2 * N, :] + hp_ref[...])
        z = jnp.dot(q, w_ref[...], preferred_element_type=jnp.float32) + b_ref[...]
        o_ref[...] = jax.nn.sigmoid(z)

    return pl.pallas_call(
        body,
        out_shape=jax.ShapeDtypeStruct((N, OUTP), jnp.float32),
    )(partials, hp, degp, W, b)


def kernel(x, edge_index, W1, b1, g1, be1, W2, b2, g2, be2, W3, b3):
    src = edge_index[0]
    dst = edge_index[1]
    pad = EPAD - E
    src3 = jnp.concatenate([src, jnp.zeros((pad,), jnp.int32)]).reshape(NW, KPC, CH)
    dst3 = jnp.concatenate([dst, jnp.full((pad,), PAD_DST, jnp.int32)]).reshape(NW, KPC, CH)

    W3p = jnp.pad(W3, ((0, 0), (0, OUTP - OUT)))
    b3p = jnp.pad(b3, (0, OUTP - OUT)).reshape(1, OUTP)
    b1r = b1.reshape(1, HID)
    g1r = g1.reshape(1, HID)
    be1r = be1.reshape(1, HID)
    b2r = b2.reshape(1, HID)
    g2r = g2.reshape(1, HID)
    be2r = be2.reshape(1, HID)

    degp = _degree(dst3)

    hp1 = _tc_first(x, W1, degp)
    p1 = _propagate(hp1, src3, dst3, HID)
    hp2 = _tc_mid(p1, hp1, degp, b1r, g1r, be1r, W2, HID)
    p2 = _propagate(hp2, src3, dst3, HID)
    hp3 = _tc_mid_nomm(p2, hp2, degp, b2r, g2r, be2r)
    p3 = _propagate(hp3, src3, dst3, HID)
    out = _tc_final(p3, hp3, degp, W3p, b3p)
    return out[:, :OUT]


# speedup vs baseline: 7.1841x; 1.1044x over previous
import functools

import jax
import jax.numpy as jnp
from jax import lax
from jax.experimental import pallas as pl
from jax.experimental.pallas import tpu as pltpu
from jax.experimental.pallas import tpu_sc as plsc

N = 10000
E = 320000
F_IN = 128
HID = 128
OUT = 11
OUTP = 16

NCORES = 2
NSUB = 16
NW = NCORES * NSUB
CH = 128
KPC = 80
EPAD = NW * KPC * CH
ACC_ROWS = 10048
PAD_DST = N
ZROWS = 16
NBUF = 2
NIB = 4


def _sc_mesh():
    return plsc.VectorSubcoreMesh(core_axis_name="c", subcore_axis_name="s")


def _degree(dst3):
    @functools.partial(
        pl.kernel,
        mesh=_sc_mesh(),
        out_type=jax.ShapeDtypeStruct((2 * N, HID), jnp.float32),
        scratch_types=[
            pltpu.VMEM((KPC, CH), jnp.int32),
            pltpu.VMEM((CH, HID), jnp.float32),
            pltpu.VMEM((ZROWS, HID), jnp.float32),
            pltpu.VMEM_SHARED((ACC_ROWS, HID), jnp.float32),
        ],
    )
    def kern(dst_hbm, out_hbm, didx, ones, zbuf, acc):
        c = lax.axis_index("c")
        s = lax.axis_index("s")
        wid = s * NCORES + c

        one16 = jnp.ones((16,), jnp.float32)
        zero16 = jnp.zeros((16,), jnp.float32)
        for i in range(CH):
            for j in range(HID // 16):
                ones[i, pl.ds(j * 16, 16)] = one16
        for i in range(ZROWS):
            for j in range(HID // 16):
                zbuf[i, pl.ds(j * 16, 16)] = zero16

        rps = ACC_ROWS // NSUB
        base = s * rps
        nz = rps // ZROWS
        tail = rps - nz * ZROWS
        for i in range(nz):
            pltpu.sync_copy(zbuf, acc.at[pl.ds(base + i * ZROWS, ZROWS)])
        pltpu.sync_copy(zbuf.at[pl.ds(0, tail)],
                        acc.at[pl.ds(base + nz * ZROWS, tail)])
        plsc.subcore_barrier()

        pltpu.sync_copy(dst_hbm.at[wid], didx)

        def body(k, carry):
            pltpu.sync_copy(ones, acc.at[didx.at[k]], add=True)
            return carry

        lax.fori_loop(0, KPC, body, 0)

        plsc.subcore_barrier()
        pltpu.sync_copy(acc.at[pl.ds(s * 624, 624)],
                        out_hbm.at[pl.ds(c * N + s * 624, 624)])
        @pl.when(s == 0)
        def _():
            pltpu.sync_copy(acc.at[pl.ds(9984, 16)],
                            out_hbm.at[pl.ds(c * N + 9984, 16)])

    return kern(dst3)


def _propagate(hp, src3, dst3, D):
    @functools.partial(
        pl.kernel,
        mesh=_sc_mesh(),
        out_type=jax.ShapeDtypeStruct((2 * N, D), jnp.float32),
        scratch_types=[
            pltpu.VMEM((NIB, CH), jnp.int32),
            pltpu.VMEM((KPC, CH), jnp.int32),
            pltpu.VMEM((NBUF, CH, D), jnp.float32),
            pltpu.VMEM((ZROWS, D), jnp.float32),
            pltpu.VMEM_SHARED((ACC_ROWS, D), jnp.float32),
            pltpu.SemaphoreType.DMA((NBUF,)),
            pltpu.SemaphoreType.DMA((NIB,)),
            pltpu.SemaphoreType.DMA,
        ],
    )
    def kern(hp_hbm, src_hbm, dst_hbm, out_hbm, sidx, didx, rows, zbuf, acc,
             gsem, isem, zsem):
        c = lax.axis_index("c")
        s = lax.axis_index("s")
        wid = s * NCORES + c

        zero16 = jnp.zeros((16,), jnp.float32)
        for i in range(ZROWS):
            for j in range(D // 16):
                zbuf[i, pl.ds(j * 16, 16)] = zero16

        rps = ACC_ROWS // NSUB
        base = s * rps
        nz = rps // ZROWS
        for i in range(nz):
            pltpu.async_copy(zbuf, acc.at[pl.ds(base + i * ZROWS, ZROWS)], zsem)
        pltpu.async_copy(zbuf.at[pl.ds(0, rps - nz * ZROWS)],
                         acc.at[pl.ds(base + nz * ZROWS, rps - nz * ZROWS)], zsem)
        pltpu.sync_copy(dst_hbm.at[wid], didx)
        for i in range(nz):
            pltpu.make_async_copy(zbuf, acc.at[pl.ds(base, ZROWS)], zsem).wait()
        pltpu.make_async_copy(zbuf.at[pl.ds(0, rps - nz * ZROWS)],
                              acc.at[pl.ds(base, rps - nz * ZROWS)], zsem).wait()
        plsc.subcore_barrier()

        for r in range(NIB):
            pltpu.async_copy(src_hbm.at[pl.ds(wid * KPC + r, 1)],
                             sidx.at[pl.ds(r, 1)], isem.at[r])
        for b in range(NBUF):
            pltpu.make_async_copy(src_hbm.at[pl.ds(0, 1)],
                                  sidx.at[pl.ds(b, 1)], isem.at[b]).wait()
            pltpu.async_copy(hp_hbm.at[sidx.at[b]], rows.at[b], gsem.at[b])

        def body(k, carry):
            b = lax.rem(k, NBUF)
            ib = lax.rem(k, NIB)
            pltpu.make_async_copy(hp_hbm.at[pl.ds(0, CH)], rows.at[b],
                                  gsem.at[b]).wait()
            pltpu.sync_copy(rows.at[b], acc.at[didx.at[k]], add=True)
            kn = k + NBUF

            @pl.when(kn < KPC)
            def _():
                inx = lax.rem(kn, NIB)
                pltpu.make_async_copy(src_hbm.at[pl.ds(0, 1)],
                                      sidx.at[pl.ds(inx, 1)], isem.at[inx]).wait()
                pltpu.async_copy(hp_hbm.at[sidx.at[inx]], rows.at[b],
                                 gsem.at[b])

            kp = k + NIB

            @pl.when(kp < KPC)
            def _():
                pltpu.async_copy(src_hbm.at[pl.ds(wid * KPC + kp, 1)],
                                 sidx.at[pl.ds(ib, 1)], isem.at[ib])

            return carry

        lax.fori_loop(0, KPC, body, 0)

        plsc.subcore_barrier()
        pltpu.sync_copy(acc.at[pl.ds(s * 624, 624)],
                        out_hbm.at[pl.ds(c * N + s * 624, 624)])
        @pl.when(s == 0)
        def _():
            pltpu.sync_copy(acc.at[pl.ds(9984, 16)],
                            out_hbm.at[pl.ds(c * N + 9984, 16)])

    return kern(hp, src3, dst3)


def _dinv_from(degp_ref):
    deg = degp_ref[0:N, 0] + degp_ref[N:2 * N, 0] + 1.0
    return lax.rsqrt(deg)


def _tc_first(x, W1, degp):
    def body(x_ref, w_ref, degp_ref, o_ref):
        dinv = _dinv_from(degp_ref)[:, None]
        h = jnp.dot(x_ref[...], w_ref[...], preferred_element_type=jnp.float32)
        o_ref[...] = h * dinv

    return pl.pallas_call(
        body,
        out_shape=jax.ShapeDtypeStruct((N, HID), jnp.float32),
    )(x, W1, degp)


def _tc_mid(partials, hp, degp, b, g, be, Wn, Dn):
    def body(p_ref, hp_ref, degp_ref, b_ref, g_ref, be_ref, w_ref, o_ref):
        dinv = _dinv_from(degp_ref)[:, None]
        z = dinv * (p_ref[0:N, :] + p_ref[N:2 * N, :] + hp_ref[...]) + b_ref[...]
        mu = jnp.mean(z, axis=0, keepdims=True)
        var = jnp.mean((z - mu) * (z - mu), axis=0, keepdims=True)
        zn = (z - mu) * lax.rsqrt(var + 1e-5) * g_ref[...] + be_ref[...]
        h = jnp.maximum(zn, 0.0)
        o_ref[...] = jnp.dot(h, w_ref[...], preferred_element_type=jnp.float32) * dinv

    return pl.pallas_call(
        body,
        out_shape=jax.ShapeDtypeStruct((N, Dn), jnp.float32),
    )(partials, hp, degp, b, g, be, Wn)


def _tc_mid_nomm(partials, hp, degp, b, g, be):
    def body(p_ref, hp_ref, degp_ref, b_ref, g_ref, be_ref, o_ref):
        dinv = _dinv_from(degp_ref)[:, None]
        z = dinv * (p_ref[0:N, :] + p_ref[N:2 * N, :] + hp_ref[...]) + b_ref[...]
        mu = jnp.mean(z, axis=0, keepdims=True)
        var = jnp.mean((z - mu) * (z - mu), axis=0, keepdims=True)
        zn = (z - mu) * lax.rsqrt(var + 1e-5) * g_ref[...] + be_ref[...]
        o_ref[...] = jnp.maximum(zn, 0.0) * dinv

    return pl.pallas_call(
        body,
        out_shape=jax.ShapeDtypeStruct((N, HID), jnp.float32),
    )(partials, hp, degp, b, g, be)


def _tc_final(partials, hp, degp, W, b):
    def body(p_ref, hp_ref, degp_ref, w_ref, b_ref, o_ref):
        dinv = _dinv_from(degp_ref)[:, None]
        q = dinv * (p_ref[0:N, :] + p_ref[N:2 * N, :] + hp_ref[...])
        z = jnp.dot(q, w_ref[...], preferred_element_type=jnp.float32) + b_ref[...]
        o_ref[...] = jax.nn.sigmoid(z)

    return pl.pallas_call(
        body,
        out_shape=jax.ShapeDtypeStruct((N, OUTP), jnp.float32),
    )(partials, hp, degp, W, b)


def kernel(x, edge_index, W1, b1, g1, be1, W2, b2, g2, be2, W3, b3):
    src = edge_index[0]
    dst = edge_index[1]
    pad = EPAD - E
    src3 = jnp.concatenate([src, jnp.zeros((pad,), jnp.int32)]).reshape(NW * KPC, CH)
    dst3 = jnp.concatenate([dst, jnp.full((pad,), PAD_DST, jnp.int32)]).reshape(NW, KPC, CH)

    W3p = jnp.pad(W3, ((0, 0), (0, OUTP - OUT)))
    b3p = jnp.pad(b3, (0, OUTP - OUT)).reshape(1, OUTP)
    b1r = b1.reshape(1, HID)
    g1r = g1.reshape(1, HID)
    be1r = be1.reshape(1, HID)
    b2r = b2.reshape(1, HID)
    g2r = g2.reshape(1, HID)
    be2r = be2.reshape(1, HID)

    degp = _degree(dst3)

    hp1 = _tc_first(x, W1, degp)
    p1 = _propagate(hp1, src3, dst3, HID)
    hp2 = _tc_mid(p1, hp1, degp, b1r, g1r, be1r, W2, HID)
    p2 = _propagate(hp2, src3, dst3, HID)
    hp3 = _tc_mid_nomm(p2, hp2, degp, b2r, g2r, be2r)
    p3 = _propagate(hp3, src3, dst3, HID)
    out = _tc_final(p3, hp3, degp, W3p, b3p)
    return out[:, :OUT]


# speedup vs baseline: 7.2212x; 1.0052x over previous
import functools

import jax
import jax.numpy as jnp
from jax import lax
from jax.experimental import pallas as pl
from jax.experimental.pallas import tpu as pltpu
from jax.experimental.pallas import tpu_sc as plsc

N = 10000
E = 320000
F_IN = 128
HID = 128
OUT = 11
OUTP = 16

NCORES = 2
NSUB = 16
NW = NCORES * NSUB
CH = 128
KPC = 80
EPAD = NW * KPC * CH
ACC_ROWS = 10016
PAD_DST = N
ZROWS = 16
ZROWSP = 4
NBUF = 3
NIB = 4


def _sc_mesh():
    return plsc.VectorSubcoreMesh(core_axis_name="c", subcore_axis_name="s")


def _degree(dst3):
    @functools.partial(
        pl.kernel,
        mesh=_sc_mesh(),
        out_type=jax.ShapeDtypeStruct((2 * N, HID), jnp.float32),
        scratch_types=[
            pltpu.VMEM((KPC, CH), jnp.int32),
            pltpu.VMEM((CH, HID), jnp.float32),
            pltpu.VMEM((ZROWS, HID), jnp.float32),
            pltpu.VMEM_SHARED((ACC_ROWS, HID), jnp.float32),
        ],
    )
    def kern(dst_hbm, out_hbm, didx, ones, zbuf, acc):
        c = lax.axis_index("c")
        s = lax.axis_index("s")
        wid = s * NCORES + c

        one16 = jnp.ones((16,), jnp.float32)
        zero16 = jnp.zeros((16,), jnp.float32)
        for i in range(CH):
            for j in range(HID // 16):
                ones[i, pl.ds(j * 16, 16)] = one16
        for i in range(ZROWS):
            for j in range(HID // 16):
                zbuf[i, pl.ds(j * 16, 16)] = zero16

        rps = ACC_ROWS // NSUB
        base = s * rps
        nz = rps // ZROWS
        tail = rps - nz * ZROWS
        for i in range(nz):
            pltpu.sync_copy(zbuf, acc.at[pl.ds(base + i * ZROWS, ZROWS)])
        pltpu.sync_copy(zbuf.at[pl.ds(0, tail)],
                        acc.at[pl.ds(base + nz * ZROWS, tail)])
        plsc.subcore_barrier()

        pltpu.sync_copy(dst_hbm.at[wid], didx)

        def body(k, carry):
            pltpu.sync_copy(ones, acc.at[didx.at[k]], add=True)
            return carry

        lax.fori_loop(0, KPC, body, 0)

        plsc.subcore_barrier()
        pltpu.sync_copy(acc.at[pl.ds(s * 624, 624)],
                        out_hbm.at[pl.ds(c * N + s * 624, 624)])
        @pl.when(s == 0)
        def _():
            pltpu.sync_copy(acc.at[pl.ds(9984, 16)],
                            out_hbm.at[pl.ds(c * N + 9984, 16)])

    return kern(dst3)


def _propagate(hp, eidx3, D):
    @functools.partial(
        pl.kernel,
        mesh=_sc_mesh(),
        out_type=jax.ShapeDtypeStruct((2 * N, D), jnp.float32),
        scratch_types=[
            pltpu.VMEM((NIB, 2, CH), jnp.int32),
            pltpu.VMEM((NBUF, CH, D), jnp.float32),
            pltpu.VMEM((ZROWSP, D), jnp.float32),
            pltpu.VMEM_SHARED((ACC_ROWS, D), jnp.float32),
            pltpu.SemaphoreType.DMA((NBUF,)),
            pltpu.SemaphoreType.DMA((NIB,)),
            pltpu.SemaphoreType.DMA,
        ],
    )
    def kern(hp_hbm, eidx_hbm, out_hbm, idx, rows, zbuf, acc,
             gsem, isem, zsem):
        c = lax.axis_index("c")
        s = lax.axis_index("s")
        wid = s * NCORES + c

        zero16 = jnp.zeros((16,), jnp.float32)
        for i in range(ZROWSP):
            for j in range(D // 16):
                zbuf[i, pl.ds(j * 16, 16)] = zero16

        rps = ACC_ROWS // NSUB
        base = s * rps
        nz = rps // ZROWSP
        for i in range(nz):
            pltpu.async_copy(zbuf, acc.at[pl.ds(base + i * ZROWSP, ZROWSP)], zsem)
        pltpu.async_copy(zbuf.at[pl.ds(0, rps - nz * ZROWSP)],
                         acc.at[pl.ds(base + nz * ZROWSP, rps - nz * ZROWSP)], zsem)
        for i in range(nz):
            pltpu.make_async_copy(zbuf, acc.at[pl.ds(base, ZROWSP)], zsem).wait()
        pltpu.make_async_copy(zbuf.at[pl.ds(0, rps - nz * ZROWSP)],
                              acc.at[pl.ds(base, rps - nz * ZROWSP)], zsem).wait()
        plsc.subcore_barrier()

        for r in range(NIB):
            pltpu.async_copy(eidx_hbm.at[wid, pl.ds(r, 1)],
                             idx.at[pl.ds(r, 1)], isem.at[r])
        for b in range(NBUF):
            pltpu.make_async_copy(eidx_hbm.at[0, pl.ds(0, 1)],
                                  idx.at[pl.ds(b, 1)], isem.at[b]).wait()
            pltpu.async_copy(hp_hbm.at[idx.at[b, 0]], rows.at[b], gsem.at[b])

        def body(k, carry):
            b = lax.rem(k, NBUF)
            ib = lax.rem(k, NIB)
            pltpu.make_async_copy(hp_hbm.at[pl.ds(0, CH)], rows.at[b],
                                  gsem.at[b]).wait()
            pltpu.sync_copy(rows.at[b], acc.at[idx.at[ib, 1]], add=True)
            kn = k + NBUF

            @pl.when(kn < KPC)
            def _():
                inx = lax.rem(kn, NIB)
                pltpu.make_async_copy(eidx_hbm.at[0, pl.ds(0, 1)],
                                      idx.at[pl.ds(inx, 1)], isem.at[inx]).wait()
                pltpu.async_copy(hp_hbm.at[idx.at[inx, 0]], rows.at[b],
                                 gsem.at[b])

            kp = k + NIB

            @pl.when(kp < KPC)
            def _():
                pltpu.async_copy(eidx_hbm.at[wid, pl.ds(kp, 1)],
                                 idx.at[pl.ds(ib, 1)], isem.at[ib])

            return carry

        lax.fori_loop(0, KPC, body, 0)

        plsc.subcore_barrier()
        pltpu.sync_copy(acc.at[pl.ds(s * 624, 624)],
                        out_hbm.at[pl.ds(c * N + s * 624, 624)])
        @pl.when(s == 0)
        def _():
            pltpu.sync_copy(acc.at[pl.ds(9984, 16)],
                            out_hbm.at[pl.ds(c * N + 9984, 16)])

    return kern(hp, eidx3)


def _dinv_from(degp_ref):
    deg = degp_ref[0:N, 0] + degp_ref[N:2 * N, 0] + 1.0
    return lax.rsqrt(deg)


def _tc_first(x, W1, degp):
    def body(x_ref, w_ref, degp_ref, o_ref):
        dinv = _dinv_from(degp_ref)[:, None]
        h = jnp.dot(x_ref[...], w_ref[...], preferred_element_type=jnp.float32)
        o_ref[...] = h * dinv

    return pl.pallas_call(
        body,
        out_shape=jax.ShapeDtypeStruct((N, HID), jnp.float32),
    )(x, W1, degp)


def _tc_mid(partials, hp, degp, b, g, be, Wn, Dn):
    def body(p_ref, hp_ref, degp_ref, b_ref, g_ref, be_ref, w_ref, o_ref):
        dinv = _dinv_from(degp_ref)[:, None]
        z = dinv * (p_ref[0:N, :] + p_ref[N:2 * N, :] + hp_ref[...]) + b_ref[...]
        mu = jnp.mean(z, axis=0, keepdims=True)
        var = jnp.mean((z - mu) * (z - mu), axis=0, keepdims=True)
        zn = (z - mu) * lax.rsqrt(var + 1e-5) * g_ref[...] + be_ref[...]
        h = jnp.maximum(zn, 0.0)
        o_ref[...] = jnp.dot(h, w_ref[...], preferred_element_type=jnp.float32) * dinv

    return pl.pallas_call(
        body,
        out_shape=jax.ShapeDtypeStruct((N, Dn), jnp.float32),
    )(partials, hp, degp, b, g, be, Wn)


def _tc_mid_nomm(partials, hp, degp, b, g, be):
    def body(p_ref, hp_ref, degp_ref, b_ref, g_ref, be_ref, o_ref):
        dinv = _dinv_from(degp_ref)[:, None]
        z = dinv * (p_ref[0:N, :] + p_ref[N:2 * N, :] + hp_ref[...]) + b_ref[...]
        mu = jnp.mean(z, axis=0, keepdims=True)
        var = jnp.mean((z - mu) * (z - mu), axis=0, keepdims=True)
        zn = (z - mu) * lax.rsqrt(var + 1e-5) * g_ref[...] + be_ref[...]
        o_ref[...] = jnp.maximum(zn, 0.0) * dinv

    return pl.pallas_call(
        body,
        out_shape=jax.ShapeDtypeStruct((N, HID), jnp.float32),
    )(partials, hp, degp, b, g, be)


def _tc_final(partials, hp, degp, W, b):
    def body(p_ref, hp_ref, degp_ref, w_ref, b_ref, o_ref):
        dinv = _dinv_from(degp_ref)[:, None]
        q = dinv * (p_ref[0:N, :] + p_ref[N:2 * N, :] + hp_ref[...])
        z = jnp.dot(q, w_ref[...], preferred_element_type=jnp.float32) + b_ref[...]
        o_ref[...] = jax.nn.sigmoid(z)

    return pl.pallas_call(
        body,
        out_shape=jax.ShapeDtypeStruct((N, OUTP), jnp.float32),
    )(partials, hp, degp, W, b)


def kernel(x, edge_index, W1, b1, g1, be1, W2, b2, g2, be2, W3, b3):
    src = edge_index[0]
    dst = edge_index[1]
    pad = EPAD - E
    srcp = jnp.concatenate([src, jnp.zeros((pad,), jnp.int32)]).reshape(NW, KPC, CH)
    dst3 = jnp.concatenate([dst, jnp.full((pad,), PAD_DST, jnp.int32)]).reshape(NW, KPC, CH)
    eidx3 = jnp.stack([srcp, dst3], axis=2)

    W3p = jnp.pad(W3, ((0, 0), (0, OUTP - OUT)))
    b3p = jnp.pad(b3, (0, OUTP - OUT)).reshape(1, OUTP)
    b1r = b1.reshape(1, HID)
    g1r = g1.reshape(1, HID)
    be1r = be1.reshape(1, HID)
    b2r = b2.reshape(1, HID)
    g2r = g2.reshape(1, HID)
    be2r = be2.reshape(1, HID)

    degp = _degree(dst3)

    hp1 = _tc_first(x, W1, degp)
    p1 = _propagate(hp1, eidx3, HID)
    hp2 = _tc_mid(p1, hp1, degp, b1r, g1r, be1r, W2, HID)
    p2 = _propagate(hp2, eidx3, HID)
    hp3 = _tc_mid_nomm(p2, hp2, degp, b2r, g2r, be2r)
    p3 = _propagate(hp3, eidx3, HID)
    out = _tc_final(p3, hp3, degp, W3p, b3p)
    return out[:, :OUT]
